# Initial kernel scaffold; baseline (speedup 1.0000x reference)
import jax
import jax.numpy as jnp
from jax.experimental import pallas as pl


def kernel(feats, edge_index, Wq, bq, Wk, bk, Wv, bv, Wo, bo):
    raise NotImplementedError("write your pallas kernel here")


# speedup vs baseline: 1.0025x; 1.0025x over previous
import functools

import jax
import jax.numpy as jnp
import numpy as np
from jax.experimental import pallas as pl
from jax.experimental.pallas import tpu as pltpu

N = 10000
D = 128
H = 8
HD = 16

_NB = 1000


def _qkv_body(x_ref, wq_ref, wk_ref, wv_ref, b_ref, q_ref, k_ref, v_ref):
    x = x_ref[...]
    b = b_ref[...]
    q_ref[...] = jnp.dot(x, wq_ref[...].T, preferred_element_type=jnp.float32) + b[0:1, :]
    k_ref[...] = jnp.dot(x, wk_ref[...].T, preferred_element_type=jnp.float32) + b[1:2, :]
    v_ref[...] = jnp.dot(x, wv_ref[...].T, preferred_element_type=jnp.float32) + b[2:3, :]


def _qkv(feats, Wq, Wk, Wv, bqkv):
    grid = (N // _NB,)
    return pl.pallas_call(
        _qkv_body,
        grid=grid,
        in_specs=[
            pl.BlockSpec((_NB, D), lambda i: (i, 0)),
            pl.BlockSpec((D, D), lambda i: (0, 0)),
            pl.BlockSpec((D, D), lambda i: (0, 0)),
            pl.BlockSpec((D, D), lambda i: (0, 0)),
            pl.BlockSpec((3, D), lambda i: (0, 0)),
        ],
        out_specs=[
            pl.BlockSpec((_NB, D), lambda i: (i, 0)),
            pl.BlockSpec((_NB, D), lambda i: (i, 0)),
            pl.BlockSpec((_NB, D), lambda i: (i, 0)),
        ],
        out_shape=[jax.ShapeDtypeStruct((N, D), jnp.float32)] * 3,
    )(feats, Wq, Wk, Wv, bqkv)


def _outproj_body(s_ref, cnt_ref, wo_ref, bo_ref, o_ref):
    s = s_ref[...]
    cnt = jnp.maximum(cnt_ref[...], 1.0)
    mean = s / cnt
    o_ref[...] = jnp.dot(mean, wo_ref[...].T, preferred_element_type=jnp.float32) + bo_ref[...][0:1, :]


def _outproj(sums, cnt, Wo, bo):
    grid = (N // _NB,)
    return pl.pallas_call(
        _outproj_body,
        grid=grid,
        in_specs=[
            pl.BlockSpec((_NB, D), lambda i: (i, 0)),
            pl.BlockSpec((_NB, 1), lambda i: (i, 0)),
            pl.BlockSpec((D, D), lambda i: (0, 0)),
            pl.BlockSpec((1, D), lambda i: (0, 0)),
        ],
        out_specs=pl.BlockSpec((_NB, D), lambda i: (i, 0)),
        out_shape=jax.ShapeDtypeStruct((N, D), jnp.float32),
    )(sums, cnt, Wo, bo.reshape(1, D))


def kernel(feats, edge_index, Wq, bq, Wk, bk, Wv, bv, Wo, bo):
    bqkv = jnp.stack([bq, bk, bv], axis=0)
    q, k, v = _qkv(feats, Wq, Wk, Wv, bqkv)
    qh = q.reshape(N, H, HD)
    kh = k.reshape(N, H, HD)
    vh = v.reshape(N, H, HD)
    r = edge_index[:, 0]
    c = edge_index[:, 1]
    scores = jnp.sum(qh[c] * kh[r], axis=-1, keepdims=True) / np.sqrt(HD)
    smax = jax.ops.segment_max(scores, c, num_segments=N)
    smax = jnp.maximum(smax, 0.0)
    ex = jnp.exp(scores - smax[c])
    denom = jax.ops.segment_sum(ex, c, num_segments=N)
    attn = ex / (denom + jnp.exp(0.0 - smax))[c]
    oute = attn * vh[r]
    sums = jax.ops.segment_sum(oute, c, num_segments=N)
    cnt = jax.ops.segment_sum(jnp.ones((E_cnt := oute.shape[0], 1), jnp.float32), c, num_segments=N)
    out = _outproj(sums.reshape(N, D), cnt, Wo, bo)
    return out
